# Initial kernel scaffold; baseline (speedup 1.0000x reference)
import jax
import jax.numpy as jnp
from jax.experimental import pallas as pl


def kernel(x, pe, patch_w, patch_b, wqkv, bqkv, wo, bo, w1, b1, w2, b2, ln1_g, ln1_b, ln2_g, ln2_b, cls_ln_g, cls_ln_b, cls_w1, cls_b1, cls_w2, cls_b2):
    raise NotImplementedError("write your pallas kernel here")


# speedup vs baseline: 21.5604x; 21.5604x over previous
import functools
import math

import jax
import jax.numpy as jnp
from jax import lax
from jax.experimental import pallas as pl
from jax.experimental.pallas import tpu as pltpu

LN_EPS = 1e-5
NUM_HEADS = 4


def _layer_norm(x, g, b):
    mu = jnp.mean(x, axis=-1, keepdims=True)
    var = jnp.mean((x - mu) * (x - mu), axis=-1, keepdims=True)
    return (x - mu) * lax.rsqrt(var + LN_EPS) * g + b


def _gelu(x):
    return 0.5 * x * (1.0 + jnp.tanh(0.7978845608028654 * (x + 0.044715 * x * x * x)))


def _vit_kernel(
    xp_ref,
    pe_ref,
    patch_w_ref, patch_b_ref,
    wqkv_ref, bqkv_ref,
    wo_ref, bo_ref,
    w1_ref, b1_ref,
    w2_ref, b2_ref,
    ln1_g_ref, ln1_b_ref,
    ln2_g_ref, ln2_b_ref,
    cls_ln_g_ref, cls_ln_b_ref,
    cls_w1_ref, cls_b1_ref,
    cls_w2_ref, cls_b2_ref,
    out_ref,
    *, num_layers, seq_len, d_model, block_b, group_g, scale,
):
    L, H, S, D = num_layers, NUM_HEADS, seq_len, d_model
    BB, G = block_b, group_g
    Dh = D // H
    GS = G * S
    M = H * GS
    NG = BB // G

    row_d = lax.broadcasted_iota(jnp.int32, (M, D), 0)
    col_d = lax.broadcasted_iota(jnp.int32, (M, D), 1)
    head_mask = (col_d // Dh == row_d // GS).astype(jnp.float32)
    row_s = lax.broadcasted_iota(jnp.int32, (M, GS), 0)
    col_s = lax.broadcasted_iota(jnp.int32, (M, GS), 1)
    neg_mask = jnp.where((row_s // S) % G == col_s // S, 0.0, -1e30)

    h = (jnp.dot(xp_ref[...], patch_w_ref[...],
                 preferred_element_type=jnp.float32)
         + patch_b_ref[...] + pe_ref[...])

    for l in range(L):
        qkv = (jnp.dot(h, wqkv_ref[l], preferred_element_type=jnp.float32)
               + bqkv_ref[l])
        q = qkv[:, 0:D]
        k = qkv[:, D:2 * D]
        v = qkv[:, 2 * D:3 * D]

        attn_parts = []
        for g in range(NG):
            sl = slice(g * GS, (g + 1) * GS)
            qg, kg, vg = q[sl], k[sl], v[sl]
            qh = jnp.concatenate([qg] * H, axis=0) * head_mask
            s = lax.dot_general(qh, kg, (((1,), (1,)), ((), ())),
                                preferred_element_type=jnp.float32)
            s = s * scale + neg_mask
            m = jnp.max(s, axis=-1, keepdims=True)
            p = jnp.exp(s - m)
            denom = jnp.sum(p, axis=-1, keepdims=True)
            p = p * pl.reciprocal(denom, approx=True)
            pv = jnp.dot(p, vg, preferred_element_type=jnp.float32)
            pvm = pv * head_mask
            ag = pvm[0:GS]
            for hh in range(1, H):
                ag = ag + pvm[hh * GS:(hh + 1) * GS]
            attn_parts.append(ag)
        attn = (attn_parts[0] if NG == 1
                else jnp.concatenate(attn_parts, axis=0))

        o = jnp.dot(attn, wo_ref[l], preferred_element_type=jnp.float32) + bo_ref[l]
        h = _layer_norm(h + o, ln1_g_ref[l], ln1_b_ref[l])

        ff = _gelu(jnp.dot(h, w1_ref[l], preferred_element_type=jnp.float32)
                   + b1_ref[l])
        ff = jnp.dot(ff, w2_ref[l], preferred_element_type=jnp.float32) + b2_ref[l]
        h = _layer_norm(h + ff, ln2_g_ref[l], ln2_b_ref[l])

    pooled = jnp.mean(h.reshape(BB, S, D), axis=1)
    z = _layer_norm(pooled, cls_ln_g_ref[...], cls_ln_b_ref[...])
    z = _gelu(jnp.dot(z, cls_w1_ref[...], preferred_element_type=jnp.float32)
              + cls_b1_ref[...])
    logits = (jnp.dot(z, cls_w2_ref[...], preferred_element_type=jnp.float32)
              + cls_b2_ref[...])
    out_ref[...] = logits.astype(out_ref.dtype)


def kernel(x, pe, patch_w, patch_b, wqkv, bqkv, wo, bo, w1, b1, w2, b2,
           ln1_g, ln1_b, ln2_g, ln2_b, cls_ln_g, cls_ln_b,
           cls_w1, cls_b1, cls_w2, cls_b2):
    B, C_in, Himg, Wimg = x.shape
    S, D = pe.shape
    P2 = patch_w.shape[0]
    L = wqkv.shape[0]
    F = w1.shape[2]
    C = cls_w2.shape[1]
    H = NUM_HEADS
    Dh = D // H
    P = int(round(math.sqrt(P2 // C_in)))
    hp, wp = Himg // P, Wimg // P

    G = 16
    BB = 128
    if B % BB != 0:
        BB = G
    R = BB * S

    xp = (x.reshape(B, C_in, hp, P, wp, P)
           .transpose(0, 2, 4, 1, 3, 5)
           .reshape(B * S, P2))
    pe_rep = jnp.tile(pe, (BB, 1))

    flops_img = (2 * S * P2 * D
                 + L * (2 * S * D * 3 * D + 2 * (H * S) * (G * S) * D
                        + 2 * (H * S) * (G * S) * D + 2 * S * D * D
                        + 2 * S * D * F + 2 * S * F * D)
                 + 2 * D * D + 2 * D * C)
    trans_img = L * (H * S * G * S + S * F) + D
    cost = pl.CostEstimate(
        flops=B * flops_img,
        transcendentals=B * trans_img,
        bytes_accessed=4 * (B * S * P2 + B * C) + 4 * 100000)

    def _f2(shape):
        return pl.BlockSpec(shape, lambda b: (0, 0))

    def _f3(shape):
        return pl.BlockSpec(shape, lambda b: (0, 0, 0))

    in_specs = [
        pl.BlockSpec((R, P2), lambda b: (b, 0)),
        _f2((R, D)),
        _f2((P2, D)), _f2((1, D)),
        _f3((L, D, 3 * D)), _f3((L, 1, 3 * D)),
        _f3((L, D, D)), _f3((L, 1, D)),
        _f3((L, D, F)), _f3((L, 1, F)),
        _f3((L, F, D)), _f3((L, 1, D)),
        _f3((L, 1, D)), _f3((L, 1, D)),
        _f3((L, 1, D)), _f3((L, 1, D)),
        _f2((1, D)), _f2((1, D)),
        _f2((D, D)), _f2((1, D)),
        _f2((D, C)), _f2((1, C)),
    ]

    out = pl.pallas_call(
        functools.partial(
            _vit_kernel,
            num_layers=L, seq_len=S, d_model=D,
            block_b=BB, group_g=G, scale=1.0 / math.sqrt(Dh)),
        grid=(B // BB,),
        in_specs=in_specs,
        out_specs=pl.BlockSpec((BB, C), lambda b: (b, 0)),
        out_shape=jax.ShapeDtypeStruct((B, C), jnp.float32),
        compiler_params=pltpu.CompilerParams(
            dimension_semantics=("parallel",)),
        cost_estimate=cost,
    )(xp, pe_rep,
      patch_w, patch_b, wqkv, bqkv, wo, bo, w1, b1, w2, b2,
      ln1_g, ln1_b, ln2_g, ln2_b, cls_ln_g, cls_ln_b,
      cls_w1, cls_b1, cls_w2, cls_b2)

    return out
